# baseline (speedup 1.0000x reference)
import functools

import jax
import jax.numpy as jnp
from jax import lax
from jax.experimental import pallas as pl
from jax.experimental.pallas import tpu as pltpu
from jax.experimental.pallas import tpu_sc as plsc

EMBED_DIM = 64
SEQ = 200
BATCH = 4096

_info = plsc.get_sparse_core_info()
_NC = _info.num_cores
_NS = _info.num_subcores
_NW = _NC * _NS

_BB = BATCH // _NW
_ET = EMBED_DIM // 8
_J = 2
_JR = _BB // _J


def _make_embed():
    mesh = plsc.VectorSubcoreMesh(core_axis_name="c", subcore_axis_name="s")

    @functools.partial(
        pl.kernel,
        mesh=mesh,
        out_type=jax.ShapeDtypeStruct((SEQ, _ET, _NW, 8, _BB), jnp.float32),
        scratch_types=[
            pltpu.VMEM((2, _BB), jnp.int32),
            pltpu.VMEM((2, _BB, EMBED_DIM), jnp.float32),
            pltpu.VMEM((2, _ET, 8, _BB), jnp.float32),
        ]
        + [pltpu.SemaphoreType.DMA] * 6,
        compiler_params=pltpu.CompilerParams(
            use_tc_tiling_on_sc=False, needs_layout_passes=False
        ),
    )
    def embed(table_hbm, idx_hbm, out_hbm, idx_v, rows_v, tr_v, *sems):
        sem_i = sems[0:2]
        sem_g = sems[2:4]
        sem_o = sems[4:6]
        wid = lax.axis_index("s") * _NC + lax.axis_index("c")
        b0 = wid * _BB

        lanes = [lax.iota(jnp.int32, 16) + (16 * k) for k in range(_BB // 16)]

        def start_idx(t, s):
            pltpu.async_copy(
                idx_hbm.at[t].at[pl.ds(b0, _BB)], idx_v.at[s], sem_i[s]
            )

        def wait_idx(s):
            pltpu.make_async_copy(
                idx_hbm.at[0].at[pl.ds(b0, _BB)], idx_v.at[s], sem_i[s]
            ).wait()

        def start_gather(s):
            for j in range(_J):
                pltpu.async_copy(
                    table_hbm.at[idx_v.at[s].at[pl.ds(j * _JR, _JR)]],
                    rows_v.at[s].at[pl.ds(j * _JR, _JR)],
                    sem_g[s],
                )

        def wait_gather(s):
            for j in range(_J):
                pltpu.make_async_copy(
                    table_hbm.at[idx_v.at[s].at[pl.ds(j * _JR, _JR)]],
                    rows_v.at[s].at[pl.ds(j * _JR, _JR)],
                    sem_g[s],
                ).wait()

        def start_out(t, s):
            pltpu.async_copy(
                tr_v.at[s], out_hbm.at[t].at[:, wid], sem_o[s]
            )

        def wait_out(s):
            pltpu.make_async_copy(
                tr_v.at[s], out_hbm.at[0].at[:, wid], sem_o[s]
            ).wait()

        def transpose(s):
            iota16 = lax.iota(jnp.int32, 16)

            @plsc.parallel_loop(0, 16, step=1, unroll=2)
            def body(d):
                perm = (iota16 + d) & 15
                for m in range(EMBED_DIM // 16):
                    idx_e = perm + (16 * m)
                    idx_et = idx_e >> 3
                    idx_es = idx_e & 7
                    for k in range(_BB // 16):
                        vals = plsc.load_gather(
                            rows_v.at[s], [lanes[k], idx_e]
                        )
                        plsc.store_scatter(
                            tr_v.at[s], [idx_et, idx_es, lanes[k]], vals
                        )

        start_idx(0, 0)
        start_idx(1, 1)
        wait_idx(0)
        start_gather(0)

        def body(g, carry):
            t0 = 2 * g
            wait_gather(0)
            pl.when(g < (SEQ // 2) - 1)(lambda: start_idx(t0 + 2, 0))
            wait_idx(1)
            start_gather(1)
            pl.when(g > 0)(lambda: wait_out(0))
            transpose(0)
            start_out(t0, 0)
            wait_gather(1)
            pl.when(g < (SEQ // 2) - 1)(lambda: start_idx(t0 + 3, 1))

            def fire_even():
                wait_idx(0)
                start_gather(0)

            pl.when(g < (SEQ // 2) - 1)(fire_even)
            pl.when(g > 0)(lambda: wait_out(1))
            transpose(1)
            start_out(t0 + 1, 1)
            return carry

        lax.fori_loop(0, SEQ // 2, body, 0)

        wait_out(0)
        wait_out(1)

    return embed


def kernel(tokens, table):
    tokens_t = tokens.T
    out5 = _make_embed()(table, tokens_t)
    out = out5.transpose((2, 4, 0, 1, 3)).reshape(BATCH, SEQ, EMBED_DIM)
    return out


# speedup vs baseline: 1.0029x; 1.0029x over previous
import functools

import jax
import jax.numpy as jnp
from jax import lax
from jax.experimental import pallas as pl
from jax.experimental.pallas import tpu as pltpu
from jax.experimental.pallas import tpu_sc as plsc

EMBED_DIM = 64
SEQ = 200
BATCH = 4096

_info = plsc.get_sparse_core_info()
_NC = _info.num_cores
_NS = _info.num_subcores
_NW = _NC * _NS

_BB = BATCH // _NW
_ET = EMBED_DIM // 8
_J = 4
_JR = _BB // _J


def _make_embed():
    mesh = plsc.VectorSubcoreMesh(core_axis_name="c", subcore_axis_name="s")

    @functools.partial(
        pl.kernel,
        mesh=mesh,
        out_type=jax.ShapeDtypeStruct((SEQ, _ET, _NW, 8, _BB), jnp.float32),
        scratch_types=[
            pltpu.VMEM((2, _BB), jnp.int32),
            pltpu.VMEM((2, _BB, EMBED_DIM), jnp.float32),
            pltpu.VMEM((2, _ET, 8, _BB), jnp.float32),
        ]
        + [pltpu.SemaphoreType.DMA] * 6,
        compiler_params=pltpu.CompilerParams(
            use_tc_tiling_on_sc=False, needs_layout_passes=False
        ),
    )
    def embed(table_hbm, idx_hbm, out_hbm, idx_v, rows_v, tr_v, *sems):
        sem_i = sems[0:2]
        sem_g = sems[2:4]
        sem_o = sems[4:6]
        wid = lax.axis_index("s") * _NC + lax.axis_index("c")
        b0 = wid * _BB

        lanes = [lax.iota(jnp.int32, 16) + (16 * k) for k in range(_BB // 16)]

        def start_idx(t, s):
            pltpu.async_copy(
                idx_hbm.at[t].at[pl.ds(b0, _BB)], idx_v.at[s], sem_i[s]
            )

        def wait_idx(s):
            pltpu.make_async_copy(
                idx_hbm.at[0].at[pl.ds(b0, _BB)], idx_v.at[s], sem_i[s]
            ).wait()

        def start_gather(s):
            for j in range(_J):
                pltpu.async_copy(
                    table_hbm.at[idx_v.at[s].at[pl.ds(j * _JR, _JR)]],
                    rows_v.at[s].at[pl.ds(j * _JR, _JR)],
                    sem_g[s],
                )

        def wait_gather(s):
            for j in range(_J):
                pltpu.make_async_copy(
                    table_hbm.at[idx_v.at[s].at[pl.ds(j * _JR, _JR)]],
                    rows_v.at[s].at[pl.ds(j * _JR, _JR)],
                    sem_g[s],
                ).wait()

        def start_out(t, s):
            pltpu.async_copy(
                tr_v.at[s], out_hbm.at[t].at[:, wid], sem_o[s]
            )

        def wait_out(s):
            pltpu.make_async_copy(
                tr_v.at[s], out_hbm.at[0].at[:, wid], sem_o[s]
            ).wait()

        def transpose(s):
            iota16 = lax.iota(jnp.int32, 16)

            @plsc.parallel_loop(0, 16, step=1, unroll=2)
            def body(d):
                perm = (iota16 + d) & 15
                for m in range(EMBED_DIM // 16):
                    idx_e = perm + (16 * m)
                    idx_et = idx_e >> 3
                    idx_es = idx_e & 7
                    for k in range(_BB // 16):
                        vals = plsc.load_gather(
                            rows_v.at[s], [lanes[k], idx_e]
                        )
                        plsc.store_scatter(
                            tr_v.at[s], [idx_et, idx_es, lanes[k]], vals
                        )

        start_idx(0, 0)
        start_idx(1, 1)
        wait_idx(0)
        start_gather(0)

        def body(g, carry):
            t0 = 2 * g
            wait_gather(0)
            pl.when(g < (SEQ // 2) - 1)(lambda: start_idx(t0 + 2, 0))
            wait_idx(1)
            start_gather(1)
            pl.when(g > 0)(lambda: wait_out(0))
            transpose(0)
            start_out(t0, 0)
            wait_gather(1)
            pl.when(g < (SEQ // 2) - 1)(lambda: start_idx(t0 + 3, 1))

            def fire_even():
                wait_idx(0)
                start_gather(0)

            pl.when(g < (SEQ // 2) - 1)(fire_even)
            pl.when(g > 0)(lambda: wait_out(1))
            transpose(1)
            start_out(t0 + 1, 1)
            return carry

        lax.fori_loop(0, SEQ // 2, body, 0)

        wait_out(0)
        wait_out(1)

    return embed


def kernel(tokens, table):
    tokens_t = tokens.T
    out5 = _make_embed()(table, tokens_t)
    out = out5.transpose((2, 4, 0, 1, 3)).reshape(BATCH, SEQ, EMBED_DIM)
    return out
